# Initial kernel scaffold; baseline (speedup 1.0000x reference)
import jax
import jax.numpy as jnp
from jax.experimental import pallas as pl


def kernel(x, route, n_routes):
    raise NotImplementedError("write your pallas kernel here")


# speedup vs baseline: 2.5234x; 2.5234x over previous
import functools

import jax
import jax.numpy as jnp
from jax import lax
from jax.experimental import pallas as pl
from jax.experimental.pallas import tpu as pltpu
from jax.experimental.pallas import tpu_sc as plsc


@functools.cache
def _route_copy(N, D):
  info = plsc.get_sparse_core_info()
  NC, NS, L = info.num_cores, info.num_subcores, info.num_lanes
  NW = NC * NS
  assert N % NW == 0 and D % L == 0
  rows_w = N // NW
  CHUNK = 32
  assert rows_w % CHUNK == 0 and CHUNK % L == 0
  n_chunks = rows_w // CHUNK
  mesh = plsc.VectorSubcoreMesh(core_axis_name="c", subcore_axis_name="s")

  @functools.partial(
      pl.kernel,
      mesh=mesh,
      out_type=jax.ShapeDtypeStruct((N * D,), jnp.float32),
      scratch_types=[
          pltpu.VMEM((CHUNK * D,), jnp.float32),
          pltpu.VMEM((CHUNK * D,), jnp.float32),
          pltpu.VMEM((rows_w,), jnp.int32),
          pltpu.VMEM((L,), jnp.int32),
          pltpu.SemaphoreType.DMA,
          pltpu.SemaphoreType.DMA,
          pltpu.SemaphoreType.DMA,
          pltpu.SemaphoreType.DMA,
      ],
      compiler_params=pltpu.CompilerParams(needs_layout_passes=False),
  )
  def run(x_hbm, route_hbm, nr_hbm, out_hbm, buf0, buf1, route_v, nr_v,
          si0, si1, so0, so1):
    wid = lax.axis_index("s") * NC + lax.axis_index("c")
    base = wid * rows_w
    pltpu.sync_copy(route_hbm.at[pl.ds(base, rows_w)], route_v)
    pltpu.sync_copy(nr_hbm, nr_v)
    nr = nr_v[...]
    bufs = (buf0, buf1)
    si = (si0, si1)
    so = (so0, so1)
    lane = lax.iota(jnp.int32, L)

    def start_in(g):
      src = x_hbm.at[pl.ds((base + g * CHUNK) * D, CHUNK * D)]
      return pltpu.async_copy(src, bufs[g % 2], si[g % 2])

    in_h = {0: start_in(0)}
    out_h = {}
    for g in range(n_chunks):
      b = g % 2
      if g + 1 < n_chunks:
        if g - 1 in out_h:
          out_h[g - 1].wait()
        in_h[g + 1] = start_in(g + 1)
      in_h[g].wait()
      buf = bufs[b]

      for k in range(CHUNK // L):
        v = route_v[pl.ds(g * CHUNK + k * L, L)]
        bad = ((v < 0) | (v >= nr)).astype(jnp.int32)
        n_bad = jnp.sum(bad)

        @pl.when(n_bad > 0)
        def _fix(bad=bad, buf=buf, k=k):
          bad_f = bad.astype(jnp.float32)

          def per_row(i, c):
            keep = 1.0 - jnp.sum(jnp.where(lane == i, bad_f, 0.0))
            keep_v = jnp.full((L,), keep, dtype=jnp.float32)
            off = (k * L + i) * D

            def per_vec(j, c2):
              sl = pl.ds(off + j * L, L)
              buf[sl] = buf[sl] * keep_v
              return c2

            return lax.fori_loop(0, D // L, per_vec, c)

          lax.fori_loop(0, L, per_row, 0)

      dst = out_hbm.at[pl.ds((base + g * CHUNK) * D, CHUNK * D)]
      out_h[g] = pltpu.async_copy(buf, dst, so[b])
    if n_chunks >= 2:
      out_h[n_chunks - 2].wait()
    out_h[n_chunks - 1].wait()

  return run


def kernel(x, route, n_routes):
  B, T, D = x.shape
  N = B * T
  xf = x.reshape(N * D)
  rf = route.reshape(N).astype(jnp.int32)
  nr = jnp.full((16,), n_routes, dtype=jnp.int32)
  out = _route_copy(N, D)(xf, rf, nr)
  return out.reshape(B, T, D)
